# baseline (speedup 1.0000x reference)
import functools

import jax
import jax.numpy as jnp
from jax import lax
from jax.experimental import pallas as pl
from jax.experimental.pallas import tpu as pltpu
from jax.experimental.pallas import tpu_sc as plsc

_PAD = 0
_EOS = 1

_NC = 2
_NS = 16
_NW = _NC * _NS

_CH = 512
_NBUF = 2


@functools.cache
def _make_gather(n_rows: int, table_rows: int, d: int):
    assert n_rows % _NW == 0
    per_w = n_rows // _NW
    assert per_w % _CH == 0
    n_chunks = per_w // _CH
    assert n_chunks % _NBUF == 0 and n_chunks >= 2 * _NBUF

    mesh = plsc.VectorSubcoreMesh(core_axis_name="c", subcore_axis_name="s")

    @functools.partial(
        pl.kernel,
        mesh=mesh,
        out_type=jax.ShapeDtypeStruct((n_rows, d), jnp.float32),
        compiler_params=pltpu.CompilerParams(use_tc_tiling_on_sc=False),
        scratch_types=[
            pltpu.VMEM((per_w,), jnp.int32),
            pltpu.VMEM((_NBUF, _CH, d), jnp.float32),
        ] + [pltpu.SemaphoreType.DMA] * (2 * _NBUF),
    )
    def gather_fn(table_hbm, idx_hbm, out_hbm, idx_v, rows_v, *sems):
        gsems = sems[:_NBUF]
        ssems = sems[_NBUF:]
        wid = lax.axis_index("s") * _NC + lax.axis_index("c")
        base = wid * per_w
        pltpu.sync_copy(idx_hbm.at[pl.ds(base, per_w)], idx_v)

        def g_copy(t, b):
            return pltpu.make_async_copy(
                table_hbm.at[idx_v.at[pl.ds(t * _CH, _CH)]], rows_v.at[b],
                gsems[b])

        def s_copy(t, b):
            return pltpu.make_async_copy(
                rows_v.at[b], out_hbm.at[pl.ds(base + t * _CH, _CH)],
                ssems[b])

        for b in range(_NBUF):
            g_copy(b, b).start()

        @pl.loop(0, n_chunks - _NBUF, step=_NBUF)
        def _(t0):
            for b in range(_NBUF):
                t = t0 + b
                g_copy(t, b).wait()
                s_copy(t, b).start()
                s_copy(t, b).wait()
                g_copy(t + _NBUF, b).start()

        for b in range(_NBUF):
            t = n_chunks - _NBUF + b
            g_copy(t, b).wait()
            s_copy(t, b).start()
            s_copy(t, b).wait()

    return gather_fn


def _mask_body(ids_ref, real_ref, eos_ref):
    ids = ids_ref[...]
    real_ref[...] = (ids != _PAD).astype(jnp.float32)
    eos_ref[...] = (ids == _EOS).astype(jnp.float32)


@functools.cache
def _make_masks(b: int, l: int):
    return pl.pallas_call(
        _mask_body,
        out_shape=(
            jax.ShapeDtypeStruct((b, l), jnp.float32),
            jax.ShapeDtypeStruct((b, l), jnp.float32),
        ),
    )


def kernel(char_embedding, lookup_ids):
    b, l = lookup_ids.shape
    table_rows, d = char_embedding.shape
    ids32 = lookup_ids.astype(jnp.int32)
    flat_ids = ids32.reshape(-1)
    mat = _make_gather(b * l, table_rows, d)(char_embedding, flat_ids)
    real, eos = _make_masks(b, l)(ids32)
    return mat.reshape(b, l, d), real, eos
